# baseline (speedup 1.0000x reference)
import functools

import jax
import jax.numpy as jnp
from jax import lax
from jax.experimental import pallas as pl
from jax.experimental.pallas import tpu as pltpu
from jax.experimental.pallas import tpu_sc as plsc

F32 = jnp.float32

N = 10000
D = 128
G = 64
NC = 2
NS = 16
NW = NC * NS
CHUNK = 128
N_PAD = 10112
N_OUT = 10112
ZROWS = N_PAD // NS
OROWS = N_OUT // NS


NBUF = 2
IB = 8


def _make_sc_agg(nchunk):
  assert nchunk % IB == 0 and IB % NBUF == 0 and (nchunk // IB) % 2 == 0
  nblk = nchunk // IB
  mesh = plsc.VectorSubcoreMesh(core_axis_name="c", subcore_axis_name="s")

  out_type = jax.ShapeDtypeStruct((NC, N_OUT, D), F32)
  scratch = (
      [pltpu.VMEM((IB, CHUNK), jnp.int32) for _ in range(4)]
      + [pltpu.VMEM((CHUNK, D), F32) for _ in range(NBUF)]
      + [pltpu.VMEM_SHARED((N_PAD, D), F32)]
      + [pltpu.SemaphoreType.DMA for _ in range(NBUF + 1)])

  def body(x_hbm, srcp, dstp, z128, agg_out, *rest):
    src_r = rest[0:2]
    dst_r = rest[2:4]
    rows = rest[4:4 + NBUF]
    acc = rest[4 + NBUF]
    sems = rest[5 + NBUF:5 + 2 * NBUF]
    isem = rest[5 + 2 * NBUF]
    c = lax.axis_index("c")
    s = lax.axis_index("s")
    wid = s * NC + c

    pltpu.sync_copy(z128, acc.at[pl.ds(s * ZROWS, ZROWS)])
    pltpu.sync_copy(srcp.at[wid, pl.ds(0, IB)], src_r[0])
    pltpu.sync_copy(dstp.at[wid, pl.ds(0, IB)], dst_r[0])
    plsc.subcore_barrier()
    for b in range(NBUF):
      pltpu.async_copy(x_hbm.at[src_r[0].at[b]], rows[b], sems[b])

    def process_block(blk, cur):
      nxt = 1 - cur
      base = blk * IB

      @pl.when(blk + 1 < nblk)
      def _():
        pltpu.async_copy(srcp.at[wid, pl.ds((blk + 1) * IB, IB)],
                         src_r[nxt], isem)
        pltpu.async_copy(dstp.at[wid, pl.ds((blk + 1) * IB, IB)],
                         dst_r[nxt], isem)

      for k in range(IB):
        j = base + k
        b = k % NBUF
        pltpu.make_async_copy(x_hbm.at[src_r[cur].at[k]], rows[b],
                              sems[b]).wait()
        pltpu.sync_copy(rows[b], acc.at[dst_r[cur].at[k]], add=True)
        if k < IB - NBUF:
          pltpu.async_copy(x_hbm.at[src_r[cur].at[k + NBUF]], rows[b], sems[b])
        else:
          if k == IB - NBUF:
            @pl.when(blk + 1 < nblk)
            def _():
              pltpu.make_async_copy(
                  srcp.at[wid, pl.ds((blk + 1) * IB, IB)], src_r[nxt],
                  isem).wait()
              pltpu.make_async_copy(
                  dstp.at[wid, pl.ds((blk + 1) * IB, IB)], dst_r[nxt],
                  isem).wait()

          @pl.when(blk + 1 < nblk)
          def _():
            pltpu.async_copy(x_hbm.at[src_r[nxt].at[k + NBUF - IB]], rows[b],
                             sems[b])

    @pl.loop(0, nblk, step=2)
    def _(blk):
      process_block(blk, 0)
      process_block(blk + 1, 1)

    plsc.subcore_barrier()
    pltpu.sync_copy(acc.at[pl.ds(s * OROWS, OROWS)],
                    agg_out.at[c, pl.ds(s * OROWS, OROWS)])

  return pl.kernel(body, out_type=out_type, mesh=mesh, scratch_types=scratch)


def _make_sc_cnt(nchunk):
  mesh = plsc.VectorSubcoreMesh(core_axis_name="c", subcore_axis_name="s")

  out_type = jax.ShapeDtypeStruct((NC, N_OUT, D), F32)
  scratch = [
      pltpu.VMEM((nchunk, CHUNK), jnp.int32),
      pltpu.VMEM((CHUNK, D), F32),
      pltpu.VMEM_SHARED((N_PAD, D), F32),
  ]

  def body(dstp, z16, ones_hbm, cnt_out, dst_v, ones_v, acc):
    c = lax.axis_index("c")
    s = lax.axis_index("s")
    wid = s * NC + c

    pltpu.sync_copy(z16, acc.at[pl.ds(s * ZROWS, ZROWS)])
    pltpu.sync_copy(ones_hbm, ones_v)
    pltpu.sync_copy(dstp.at[wid], dst_v)
    plsc.subcore_barrier()

    @pl.loop(0, nchunk)
    def _(j):
      pltpu.sync_copy(ones_v, acc.at[dst_v.at[j]], add=True)

    plsc.subcore_barrier()
    pltpu.sync_copy(acc.at[pl.ds(s * OROWS, OROWS)],
                    cnt_out.at[c, pl.ds(s * OROWS, OROWS)])

  return pl.kernel(body, out_type=out_type, mesh=mesh, scratch_types=scratch)


def _sc_aggregate(x, srcp, dstp):
  nchunk = srcp.shape[1]
  z128 = jnp.zeros((ZROWS, D), F32)
  return _make_sc_agg(nchunk)(x, srcp, dstp, z128)


def _sc_count(dstp):
  nchunk = dstp.shape[1]
  z16 = jnp.zeros((ZROWS, D), F32)
  ones = jnp.ones((CHUNK, D), F32)
  return _make_sc_cnt(nchunk)(dstp, z16, ones)


_BM = 2000
_NB = N // _BM


def _dotT(a, w):
  return lax.dot_general(a, w, (((1,), (1,)), ((), ())),
                         preferred_element_type=F32)


def _layer_body(has_res, *refs):
  if has_res:
    (aggp, cntp, x, wl, bl, wr, g, bn, rw, rb, o) = refs
  else:
    (aggp, cntp, x, wl, bl, wr, g, bn, o) = refs
  agg = aggp[0] + aggp[1]
  cnt = cntp[0, :, 0:1] + cntp[1, :, 0:1]
  agg = agg / jnp.maximum(cnt, 1.0)
  z = _dotT(agg, wl[...]) + bl[...] + _dotT(x[...], wr[...])
  z = jnp.maximum(z, 0.0)
  m = jnp.mean(z, axis=-1, keepdims=True)
  v = jnp.mean((z - m) * (z - m), axis=-1, keepdims=True)
  z = (z - m) * lax.rsqrt(v + 1e-5) * g[...] + bn[...]
  if has_res:
    z = z + _dotT(x[...], rw[...]) + rb[...]
  o[...] = z


def _tc_layer(aggp, cntp, x, wl, bl, wr, g, bn, rw=None, rb=None):
  has_res = rw is not None
  row = lambda i: (0, i, 0)
  mat = lambda i: (i, 0)
  full = lambda i: (0, 0)
  in_specs = [
      pl.BlockSpec((NC, _BM, D), row),
      pl.BlockSpec((NC, _BM, D), row),
      pl.BlockSpec((_BM, D), mat),
      pl.BlockSpec((D, D), full),
      pl.BlockSpec((1, D), full),
      pl.BlockSpec((D, D), full),
      pl.BlockSpec((1, D), full),
      pl.BlockSpec((1, D), full),
  ]
  args = [aggp, cntp, x, wl, bl, wr, g, bn]
  if has_res:
    in_specs += [pl.BlockSpec((D, D), full), pl.BlockSpec((1, D), full)]
    args += [rw, rb]
  return pl.pallas_call(
      functools.partial(_layer_body, has_res),
      grid=(_NB,),
      in_specs=in_specs,
      out_specs=pl.BlockSpec((_BM, D), mat),
      out_shape=jax.ShapeDtypeStruct((N, D), F32),
  )(*args)


def _final_body(aggp, cntp, x, wl, bl, wr, rw, rb, bt3,
                ws1, ws2, bs, wr1, br1, wr2, br2, wb1, bb1, wb2, bb2,
                reg_o, bin_o, acc_sum, acc_max):
  i = pl.program_id(0)

  agg = aggp[0] + aggp[1]
  cnt = cntp[0, :, 0:1] + cntp[1, :, 0:1]
  agg = agg / jnp.maximum(cnt, 1.0)
  z = _dotT(agg, wl[...]) + bl[...] + _dotT(x[...], wr[...])
  z = jnp.maximum(z, 0.0)
  x3 = z + _dotT(x[...], rw[...]) + rb[...]

  bt_col = bt3[0]

  @pl.when(i == 0)
  def _():
    acc_sum[...] = jnp.zeros((G, D), F32)
    acc_max[...] = jnp.full((G, D), -jnp.inf, F32)

  seg = lax.broadcasted_iota(jnp.int32, (_BM, G), 1)
  onehot = (bt_col == seg).astype(F32)
  acc_sum[...] += lax.dot_general(onehot, x3, (((0,), (0,)), ((), ())),
                                  preferred_element_type=F32)

  bmin = jnp.min(bt_col)
  bmax = jnp.max(bt_col)
  for g in range(G):
    @pl.when((g >= bmin) & (g <= bmax))
    def _(g=g):
      vals = jnp.where(bt_col == g, x3, -jnp.inf)
      acc_max[g:g + 1, :] = jnp.maximum(
          acc_max[g:g + 1, :], jnp.max(vals, axis=0, keepdims=True))

  @pl.when(i == _NB - 1)
  def _():
    xadd = acc_sum[...]
    xmax = acc_max[...]
    shared = _dotT(xadd, ws1[...]) + _dotT(xmax, ws2[...]) + bs[...]
    shared = jnp.maximum(shared, 0.0)
    hr = jnp.maximum(_dotT(shared, wr1[...]) + br1[...], 0.0)
    reg_o[...] = _dotT(hr, wr2[...]) + br2[...]
    hb = jnp.maximum(_dotT(shared, wb1[...]) + bb1[...], 0.0)
    bin_o[...] = _dotT(hb, wb2[...]) + bb2[...]


def _tc_final(aggp, cntp, x, wl, bl, wr, rw, rb, bt3,
              ws1, ws2, bs, wr1, br1, wr2, br2, wb1, bb1, wb2, bb2):
  row = lambda i: (0, i, 0)
  mat = lambda i: (i, 0)
  full = lambda i: (0, 0)
  w = lambda shape: pl.BlockSpec(shape, full)
  in_specs = [
      pl.BlockSpec((NC, _BM, D), row),
      pl.BlockSpec((NC, _BM, D), row),
      pl.BlockSpec((_BM, D), mat),
      w((D, D)), w((1, D)), w((D, D)), w((D, D)), w((1, D)),
      pl.BlockSpec((1, _BM, 1), lambda i: (i, 0, 0)),
      w((D, D)), w((D, D)), w((1, D)),
      w((G, D)), w((1, G)), w((8, G)), w((1, 8)),
      w((G, D)), w((1, G)), w((8, G)), w((1, 8)),
  ]
  return pl.pallas_call(
      _final_body,
      grid=(_NB,),
      in_specs=in_specs,
      out_specs=(pl.BlockSpec((G, 8), full), pl.BlockSpec((G, 8), full)),
      out_shape=(jax.ShapeDtypeStruct((G, 8), F32),
                 jax.ShapeDtypeStruct((G, 8), F32)),
      scratch_shapes=[pltpu.VMEM((G, D), F32), pltpu.VMEM((G, D), F32)],
  )(aggp, cntp, x, wl, bl, wr, rw, rb, bt3,
    ws1, ws2, bs, wr1, br1, wr2, br2, wb1, bb1, wb2, bb2)


def kernel(x, edge_index, edge_attr, batch, W1l, b1l, W1r, g1, bn1,
           W2l, b2l, W2r, g2, bn2, W3l, b3l, W3r, R1W, R1b, R2W, R2b,
           Ws, bs, Wr1, br1, Wr2, br2, Wb1, bb1, Wb2, bb2):
  E = edge_index.shape[1]
  nchunk = -(-E // (NW * CHUNK))
  nchunk = -(-nchunk // (2 * IB)) * (2 * IB)
  ep = NW * nchunk * CHUNK
  src = jnp.concatenate([edge_index[0], jnp.zeros((ep - E,), jnp.int32)])
  dst = jnp.concatenate([edge_index[1], jnp.full((ep - E,), N, jnp.int32)])
  srcp = src.reshape(NW, nchunk, CHUNK)
  dstp = dst.reshape(NW, nchunk, CHUNK)

  r = lambda v: v.reshape(1, -1)

  cntp = _sc_count(dstp)
  agg1p = _sc_aggregate(x, srcp, dstp)
  x1 = _tc_layer(agg1p, cntp, x, W1l, r(b1l), W1r, r(g1), r(bn1))
  agg2p = _sc_aggregate(x1, srcp, dstp)
  x2 = _tc_layer(agg2p, cntp, x1, W2l, r(b2l), W2r, r(g2), r(bn2), R1W, r(R1b))
  agg3p = _sc_aggregate(x2, srcp, dstp)

  bt3 = batch.reshape(_NB, _BM, 1)
  pad8 = lambda w_: jnp.concatenate(
      [w_, jnp.zeros((8 - w_.shape[0],) + w_.shape[1:], F32)], axis=0)
  padb = lambda v: jnp.concatenate(
      [v, jnp.zeros((8 - v.shape[0],), F32)]).reshape(1, 8)
  reg8, bin8 = _tc_final(
      agg3p, cntp, x2, W3l, r(b3l), W3r, R2W, r(R2b), bt3,
      Ws[:, :D], Ws[:, D:], r(bs), Wr1, r(br1), pad8(Wr2), padb(br2),
      Wb1, r(bb1), pad8(Wb2), padb(bb2))
  return (reg8[:, :1], bin8[:, :2])


# speedup vs baseline: 1.3552x; 1.3552x over previous
import functools

import jax
import jax.numpy as jnp
from jax import lax
from jax.experimental import pallas as pl
from jax.experimental.pallas import tpu as pltpu
from jax.experimental.pallas import tpu_sc as plsc

F32 = jnp.float32

N = 10000
D = 128
G = 64
NC = 2
NS = 16
NW = NC * NS
CHUNK = 128
N_PAD = 10112
N_OUT = 10112
ZROWS = N_PAD // NS
OROWS = N_OUT // NS


def _make_sc_agg(nchunk):
  mesh = plsc.VectorSubcoreMesh(core_axis_name="c", subcore_axis_name="s")

  out_type = jax.ShapeDtypeStruct((NC, N_OUT, D), F32)
  scratch = [
      pltpu.VMEM((nchunk, CHUNK), jnp.int32),
      pltpu.VMEM((nchunk, CHUNK), jnp.int32),
      pltpu.VMEM((CHUNK, D), F32),
      pltpu.VMEM_SHARED((N_PAD, D), F32),
      pltpu.SemaphoreType.DMA,
  ]

  def body(x_hbm, srcp, dstp, z128, agg_out, src_v, dst_v, rows_v, acc, sem):
    c = lax.axis_index("c")
    s = lax.axis_index("s")
    wid = s * NC + c

    pltpu.sync_copy(z128, acc.at[pl.ds(s * ZROWS, ZROWS)])
    pltpu.sync_copy(srcp.at[wid], src_v)
    pltpu.sync_copy(dstp.at[wid], dst_v)
    plsc.subcore_barrier()

    @pl.loop(0, nchunk)
    def _(j):
      pltpu.async_copy(x_hbm.at[src_v.at[j]], rows_v, sem).wait()
      pltpu.sync_copy(rows_v, acc.at[dst_v.at[j]], add=True)

    plsc.subcore_barrier()
    pltpu.sync_copy(acc.at[pl.ds(s * OROWS, OROWS)],
                    agg_out.at[c, pl.ds(s * OROWS, OROWS)])

  return pl.kernel(body, out_type=out_type, mesh=mesh, scratch_types=scratch)


def _make_sc_cnt(nchunk):
  mesh = plsc.VectorSubcoreMesh(core_axis_name="c", subcore_axis_name="s")

  out_type = jax.ShapeDtypeStruct((NC, N_OUT, D), F32)
  scratch = [
      pltpu.VMEM((nchunk, CHUNK), jnp.int32),
      pltpu.VMEM((CHUNK, D), F32),
      pltpu.VMEM_SHARED((N_PAD, D), F32),
  ]

  def body(dstp, z16, ones_hbm, cnt_out, dst_v, ones_v, acc):
    c = lax.axis_index("c")
    s = lax.axis_index("s")
    wid = s * NC + c

    pltpu.sync_copy(z16, acc.at[pl.ds(s * ZROWS, ZROWS)])
    pltpu.sync_copy(ones_hbm, ones_v)
    pltpu.sync_copy(dstp.at[wid], dst_v)
    plsc.subcore_barrier()

    @pl.loop(0, nchunk)
    def _(j):
      pltpu.sync_copy(ones_v, acc.at[dst_v.at[j]], add=True)

    plsc.subcore_barrier()
    pltpu.sync_copy(acc.at[pl.ds(s * OROWS, OROWS)],
                    cnt_out.at[c, pl.ds(s * OROWS, OROWS)])

  return pl.kernel(body, out_type=out_type, mesh=mesh, scratch_types=scratch)


def _sc_aggregate(x, srcp, dstp):
  nchunk = srcp.shape[1]
  z128 = jnp.zeros((ZROWS, D), F32)
  return _make_sc_agg(nchunk)(x, srcp, dstp, z128)


def _sc_count(dstp):
  nchunk = dstp.shape[1]
  z16 = jnp.zeros((ZROWS, D), F32)
  ones = jnp.ones((CHUNK, D), F32)
  return _make_sc_cnt(nchunk)(dstp, z16, ones)


_BM = 2000
_NB = N // _BM


def _dotT(a, w):
  return lax.dot_general(a, w, (((1,), (1,)), ((), ())),
                         preferred_element_type=F32)


def _layer_body(has_res, *refs):
  if has_res:
    (aggp, cntp, x, wl, bl, wr, g, bn, rw, rb, o) = refs
  else:
    (aggp, cntp, x, wl, bl, wr, g, bn, o) = refs
  agg = aggp[0] + aggp[1]
  cnt = cntp[0, :, 0:1] + cntp[1, :, 0:1]
  agg = agg / jnp.maximum(cnt, 1.0)
  z = _dotT(agg, wl[...]) + bl[...] + _dotT(x[...], wr[...])
  z = jnp.maximum(z, 0.0)
  m = jnp.mean(z, axis=-1, keepdims=True)
  v = jnp.mean((z - m) * (z - m), axis=-1, keepdims=True)
  z = (z - m) * lax.rsqrt(v + 1e-5) * g[...] + bn[...]
  if has_res:
    z = z + _dotT(x[...], rw[...]) + rb[...]
  o[...] = z


def _tc_layer(aggp, cntp, x, wl, bl, wr, g, bn, rw=None, rb=None):
  has_res = rw is not None
  row = lambda i: (0, i, 0)
  mat = lambda i: (i, 0)
  full = lambda i: (0, 0)
  in_specs = [
      pl.BlockSpec((NC, _BM, D), row),
      pl.BlockSpec((NC, _BM, D), row),
      pl.BlockSpec((_BM, D), mat),
      pl.BlockSpec((D, D), full),
      pl.BlockSpec((1, D), full),
      pl.BlockSpec((D, D), full),
      pl.BlockSpec((1, D), full),
      pl.BlockSpec((1, D), full),
  ]
  args = [aggp, cntp, x, wl, bl, wr, g, bn]
  if has_res:
    in_specs += [pl.BlockSpec((D, D), full), pl.BlockSpec((1, D), full)]
    args += [rw, rb]
  return pl.pallas_call(
      functools.partial(_layer_body, has_res),
      grid=(_NB,),
      in_specs=in_specs,
      out_specs=pl.BlockSpec((_BM, D), mat),
      out_shape=jax.ShapeDtypeStruct((N, D), F32),
  )(*args)


def _final_body(aggp, cntp, x, wl, bl, wr, rw, rb, bt3,
                ws1, ws2, bs, wr1, br1, wr2, br2, wb1, bb1, wb2, bb2,
                reg_o, bin_o, acc_sum, acc_max):
  i = pl.program_id(0)

  agg = aggp[0] + aggp[1]
  cnt = cntp[0, :, 0:1] + cntp[1, :, 0:1]
  agg = agg / jnp.maximum(cnt, 1.0)
  z = _dotT(agg, wl[...]) + bl[...] + _dotT(x[...], wr[...])
  z = jnp.maximum(z, 0.0)
  x3 = z + _dotT(x[...], rw[...]) + rb[...]

  bt_col = bt3[0]

  @pl.when(i == 0)
  def _():
    acc_sum[...] = jnp.zeros((G, D), F32)
    acc_max[...] = jnp.full((G, D), -jnp.inf, F32)

  seg = lax.broadcasted_iota(jnp.int32, (_BM, G), 1)
  onehot = (bt_col == seg).astype(F32)
  acc_sum[...] += lax.dot_general(onehot, x3, (((0,), (0,)), ((), ())),
                                  preferred_element_type=F32)

  bmin = jnp.min(bt_col)
  bmax = jnp.max(bt_col)
  for g in range(G):
    @pl.when((g >= bmin) & (g <= bmax))
    def _(g=g):
      vals = jnp.where(bt_col == g, x3, -jnp.inf)
      acc_max[g:g + 1, :] = jnp.maximum(
          acc_max[g:g + 1, :], jnp.max(vals, axis=0, keepdims=True))

  @pl.when(i == _NB - 1)
  def _():
    xadd = acc_sum[...]
    xmax = acc_max[...]
    shared = _dotT(xadd, ws1[...]) + _dotT(xmax, ws2[...]) + bs[...]
    shared = jnp.maximum(shared, 0.0)
    hr = jnp.maximum(_dotT(shared, wr1[...]) + br1[...], 0.0)
    reg_o[...] = _dotT(hr, wr2[...]) + br2[...]
    hb = jnp.maximum(_dotT(shared, wb1[...]) + bb1[...], 0.0)
    bin_o[...] = _dotT(hb, wb2[...]) + bb2[...]


def _tc_final(aggp, cntp, x, wl, bl, wr, rw, rb, bt3,
              ws1, ws2, bs, wr1, br1, wr2, br2, wb1, bb1, wb2, bb2):
  row = lambda i: (0, i, 0)
  mat = lambda i: (i, 0)
  full = lambda i: (0, 0)
  w = lambda shape: pl.BlockSpec(shape, full)
  in_specs = [
      pl.BlockSpec((NC, _BM, D), row),
      pl.BlockSpec((NC, _BM, D), row),
      pl.BlockSpec((_BM, D), mat),
      w((D, D)), w((1, D)), w((D, D)), w((D, D)), w((1, D)),
      pl.BlockSpec((1, _BM, 1), lambda i: (i, 0, 0)),
      w((D, D)), w((D, D)), w((1, D)),
      w((G, D)), w((1, G)), w((8, G)), w((1, 8)),
      w((G, D)), w((1, G)), w((8, G)), w((1, 8)),
  ]
  return pl.pallas_call(
      _final_body,
      grid=(_NB,),
      in_specs=in_specs,
      out_specs=(pl.BlockSpec((G, 8), full), pl.BlockSpec((G, 8), full)),
      out_shape=(jax.ShapeDtypeStruct((G, 8), F32),
                 jax.ShapeDtypeStruct((G, 8), F32)),
      scratch_shapes=[pltpu.VMEM((G, D), F32), pltpu.VMEM((G, D), F32)],
  )(aggp, cntp, x, wl, bl, wr, rw, rb, bt3,
    ws1, ws2, bs, wr1, br1, wr2, br2, wb1, bb1, wb2, bb2)


def kernel(x, edge_index, edge_attr, batch, W1l, b1l, W1r, g1, bn1,
           W2l, b2l, W2r, g2, bn2, W3l, b3l, W3r, R1W, R1b, R2W, R2b,
           Ws, bs, Wr1, br1, Wr2, br2, Wb1, bb1, Wb2, bb2):
  E = edge_index.shape[1]
  nchunk = -(-E // (NW * CHUNK))
  ep = NW * nchunk * CHUNK
  src = jnp.concatenate([edge_index[0], jnp.zeros((ep - E,), jnp.int32)])
  dst = jnp.concatenate([edge_index[1], jnp.full((ep - E,), N, jnp.int32)])
  srcp = src.reshape(NW, nchunk, CHUNK)
  dstp = dst.reshape(NW, nchunk, CHUNK)

  r = lambda v: v.reshape(1, -1)

  cntp = _sc_count(dstp)
  agg1p = _sc_aggregate(x, srcp, dstp)
  x1 = _tc_layer(agg1p, cntp, x, W1l, r(b1l), W1r, r(g1), r(bn1))
  agg2p = _sc_aggregate(x1, srcp, dstp)
  x2 = _tc_layer(agg2p, cntp, x1, W2l, r(b2l), W2r, r(g2), r(bn2), R1W, r(R1b))
  agg3p = _sc_aggregate(x2, srcp, dstp)

  bt3 = batch.reshape(_NB, _BM, 1)
  pad8 = lambda w_: jnp.concatenate(
      [w_, jnp.zeros((8 - w_.shape[0],) + w_.shape[1:], F32)], axis=0)
  padb = lambda v: jnp.concatenate(
      [v, jnp.zeros((8 - v.shape[0],), F32)]).reshape(1, 8)
  reg8, bin8 = _tc_final(
      agg3p, cntp, x2, W3l, r(b3l), W3r, R2W, r(R2b), bt3,
      Ws[:, :D], Ws[:, D:], r(bs), Wr1, r(br1), pad8(Wr2), padb(br2),
      Wb1, r(bb1), pad8(Wb2), padb(bb2))
  return (reg8[:, :1], bin8[:, :2])
